# Initial kernel scaffold; baseline (speedup 1.0000x reference)
import jax
import jax.numpy as jnp
from jax.experimental import pallas as pl


def kernel(x, pruned_idx):
    raise NotImplementedError("write your pallas kernel here")


# speedup vs baseline: 4.9613x; 4.9613x over previous
import jax
import jax.numpy as jnp
from jax import lax
from jax.experimental import pallas as pl
from jax.experimental.pallas import tpu as pltpu
from jax.experimental.pallas import tpu_sc as plsc

_NC = 2
_NS = 16
_NW = _NC * _NS
_LANES = 16
_TPL = 16


def _make_body(chunks):
    def _scatter_zeros_body(idx_hbm, out_hbm, idx_v, zeros_v, sem_idx, sem_sc):
        wid = lax.axis_index("s") * _NC + lax.axis_index("c")
        d = zeros_v.shape[1]

        idx_cp = pltpu.make_async_copy(idx_hbm.at[wid], idx_v, sem_idx)
        idx_cp.start()

        z = jnp.zeros((_LANES,), jnp.float32)
        for r in range(_TPL):
            for c in range(d // _LANES):
                zeros_v[r, pl.ds(c * _LANES, _LANES)] = z

        idx_cp.wait()
        cps = []
        for j in range(chunks):
            cp = pltpu.make_async_copy(
                zeros_v, out_hbm.at[idx_v.at[j]], sem_sc)
            cp.start()
            cps.append(cp)
        for cp in cps:
            cp.wait()
    return _scatter_zeros_body


def kernel(x, pruned_idx):
    m, d = x.shape
    p = pruned_idx.shape[0]
    idx32 = pruned_idx.astype(jnp.int32)
    chunk_rows = -(-p // (_NW * _TPL))
    pad = _NW * chunk_rows * _TPL - p
    if pad:
        idx32 = jnp.concatenate(
            [idx32, jnp.broadcast_to(idx32[:1], (pad,))])
    idx3d = idx32.reshape(_NW, chunk_rows, _TPL)

    out_ref = jax.new_ref(x)

    mesh = plsc.VectorSubcoreMesh(
        core_axis_name="c", subcore_axis_name="s",
        num_cores=_NC, num_subcores=_NS)
    scatter = pl.kernel(
        _make_body(chunk_rows),
        out_type=(),
        mesh=mesh,
        scratch_types=[
            pltpu.VMEM((chunk_rows, _TPL), jnp.int32),
            pltpu.VMEM((_TPL, d), jnp.float32),
            pltpu.SemaphoreType.DMA,
            pltpu.SemaphoreType.DMA,
        ],
    )
    scatter(idx3d, out_ref)
    return jax.freeze(out_ref)
